# baseline (speedup 1.0000x reference)
import jax
import jax.numpy as jnp
from jax import lax
from jax.experimental import pallas as pl
from jax.experimental.pallas import tpu as pltpu
from jax.experimental.pallas import tpu_sc as plsc

_K = 64
_B, _N = 128, 32768
_NC, _NS = 2, 16
_NW = _NC * _NS
_RPW = _B // _NW
_L = 16
_CH = _N // _L
_D = 16


def _butterfly(c, iota, op):
    for st in (1, 2, 4, 8):
        c = op(c, jnp.take(c, iota ^ st))
    return c


def _ladder_pass(xrow, depth):
    neg_inf = jnp.full((_L,), -jnp.inf, jnp.float32)

    def body(j, regs):
        v = xrow[pl.ds(j * _L, _L)]
        new = []
        for i in range(depth):
            hi = jnp.maximum(regs[i], v)
            v = jnp.minimum(regs[i], v)
            new.append(hi)
        return tuple(new)

    return lax.fori_loop(0, _CH, body, (neg_inf,) * depth, unroll=4)


def _select_kth(regs, iota):
    neg_inf = jnp.full((_L,), -jnp.inf, jnp.float32)
    big = jnp.int32(_L)

    def step(_, carry):
        d, t = carry
        h = neg_inf
        for i, r in enumerate(regs):
            h = jnp.where(d == jnp.int32(i), r, h)
        gmax = _butterfly(h, iota, jnp.maximum)
        lanepos = jnp.where(h == gmax, iota, big)
        minlane = _butterfly(lanepos, iota, jnp.minimum)
        d = d + jnp.where(iota == minlane, jnp.int32(1), jnp.int32(0))
        return (d, gmax)

    _, t = lax.fori_loop(0, _K, step, (jnp.zeros((_L,), jnp.int32), neg_inf))
    return t


def _count_pass(xrow, tf, iota):
    zero = jnp.zeros((_L,), jnp.int32)

    def body(j, accs):
        a_gt, a_ge = accs
        v = xrow[pl.ds(j * _L, _L)]
        a_gt = a_gt + jnp.where(v > tf, jnp.int32(1), jnp.int32(0))
        a_ge = a_ge + jnp.where(v >= tf, jnp.int32(1), jnp.int32(0))
        return (a_gt, a_ge)

    a_gt, a_ge = lax.fori_loop(0, _CH, body, (zero, zero), unroll=8)
    return _butterfly(a_gt, iota, jnp.add), _butterfly(a_ge, iota, jnp.add)


def _process_row(xrow, orow, mrow):
    iota = lax.iota(jnp.int32, _L)
    zero_i = jnp.zeros((_L,), jnp.int32)

    regs = _ladder_pass(xrow, _D)
    t_est = _select_kth(regs, iota)

    c_gt_est, c_ge_est = _count_pass(xrow, t_est, iota)
    nfb = jnp.where(c_gt_est[0] > jnp.int32(_K - 1), jnp.int32(1), jnp.int32(0))

    def fb(_, carry):
        regs64 = _ladder_pass(xrow, _K)
        tf = _select_kth(regs64, iota)
        fb_gt, fb_ge = _count_pass(xrow, tf, iota)
        return (tf, fb_gt, fb_ge)

    t, c_gt, c_ge = lax.fori_loop(0, nfb, fb, (t_est, c_gt_est, c_ge_est))

    def wmain(j, carry):
        v = xrow[pl.ds(j * _L, _L)]
        keep = v >= t
        orow[pl.ds(j * _L, _L)] = jnp.where(keep, v, jnp.float32(0.0))
        mrow[pl.ds(j * _L, _L)] = jnp.where(keep, jnp.float32(1.0), jnp.float32(0.0))
        return carry

    lax.fori_loop(0, _CH, wmain, jnp.int32(0), unroll=8)

    m_total = jnp.int32(_K) - c_gt
    nt = jnp.where(c_ge[0] > jnp.int32(_K), jnp.int32(_CH), jnp.int32(0))

    def wfix(j, used):
        v = xrow[pl.ds(j * _L, _L)]
        gt = v > t
        eq = v == t
        c = jnp.where(eq, jnp.int32(1), jnp.int32(0))
        pre = c
        for st in (1, 2, 4, 8):
            shifted = jnp.take(pre, jnp.maximum(iota - st, 0))
            pre = pre + jnp.where(iota >= st, shifted, jnp.int32(0))
        keep = jnp.logical_or(gt, jnp.logical_and(eq, (used + pre) <= m_total))
        orow[pl.ds(j * _L, _L)] = jnp.where(keep, v, jnp.float32(0.0))
        mrow[pl.ds(j * _L, _L)] = jnp.where(keep, jnp.float32(1.0), jnp.float32(0.0))
        return used + _butterfly(c, iota, jnp.add)

    lax.fori_loop(0, nt, wfix, zero_i)


def _sc_body(x_hbm, out_hbm, mask_hbm, xrow, orow, mrow):
    c = lax.axis_index("c")
    s = lax.axis_index("s")
    wid = s * _NC + c
    base = wid * _RPW

    def row_step(i, carry):
        r = base + i
        pltpu.sync_copy(x_hbm.at[r], xrow)
        _process_row(xrow, orow, mrow)
        pltpu.sync_copy(orow, out_hbm.at[r])
        pltpu.sync_copy(mrow, mask_hbm.at[r])
        return carry

    lax.fori_loop(0, _RPW, row_step, jnp.int32(0))


def kernel(x):
    mesh = plsc.VectorSubcoreMesh(core_axis_name="c", subcore_axis_name="s")
    f = pl.kernel(
        _sc_body,
        out_type=[
            jax.ShapeDtypeStruct((_B, _N), jnp.float32),
            jax.ShapeDtypeStruct((_B, _N), jnp.float32),
        ],
        mesh=mesh,
        scratch_types=[
            pltpu.VMEM((_N,), jnp.float32),
            pltpu.VMEM((_N,), jnp.float32),
            pltpu.VMEM((_N,), jnp.float32),
        ],
    )
    out, mask = f(x)
    return (out, mask)


# speedup vs baseline: 1.0670x; 1.0670x over previous
import jax
import jax.numpy as jnp
from jax import lax
from jax.experimental import pallas as pl
from jax.experimental.pallas import tpu as pltpu
from jax.experimental.pallas import tpu_sc as plsc

_K = 64
_B, _N = 128, 32768
_NC, _NS = 2, 16
_NW = _NC * _NS
_RPW = _B // _NW
_L = 16
_CH = _N // _L
_D = 16


def _butterfly(c, iota, op):
    for st in (1, 2, 4, 8):
        c = op(c, jnp.take(c, iota ^ st))
    return c


def _ladder_pass(xrow, depth):
    neg_inf = jnp.full((_L,), -jnp.inf, jnp.float32)

    def body(j, regs):
        v = xrow[pl.ds(j * _L, _L)]
        new = []
        for i in range(depth):
            hi = jnp.maximum(regs[i], v)
            v = jnp.minimum(regs[i], v)
            new.append(hi)
        return tuple(new)

    return lax.fori_loop(0, _CH, body, (neg_inf,) * depth, unroll=4)


def _select_kth(regs, iota):
    neg_inf = jnp.full((_L,), -jnp.inf, jnp.float32)
    big = jnp.int32(_L)

    def step(_, carry):
        d, t = carry
        h = neg_inf
        for i, r in enumerate(regs):
            h = jnp.where(d == jnp.int32(i), r, h)
        gmax = _butterfly(h, iota, jnp.maximum)
        lanepos = jnp.where(h == gmax, iota, big)
        minlane = _butterfly(lanepos, iota, jnp.minimum)
        d = d + jnp.where(iota == minlane, jnp.int32(1), jnp.int32(0))
        return (d, gmax)

    _, t = lax.fori_loop(0, _K, step, (jnp.zeros((_L,), jnp.int32), neg_inf))
    return t


def _count_pass(xrow, tf, iota):
    zero = jnp.zeros((_L,), jnp.int32)

    def body(j, accs):
        a_gt, a_ge = accs
        v = xrow[pl.ds(j * _L, _L)]
        a_gt = a_gt + jnp.where(v > tf, jnp.int32(1), jnp.int32(0))
        a_ge = a_ge + jnp.where(v >= tf, jnp.int32(1), jnp.int32(0))
        return (a_gt, a_ge)

    a_gt, a_ge = lax.fori_loop(0, _CH, body, (zero, zero), unroll=8)
    return _butterfly(a_gt, iota, jnp.add), _butterfly(a_ge, iota, jnp.add)


def _process_row(xrow, orow, mrow):
    iota = lax.iota(jnp.int32, _L)
    zero_i = jnp.zeros((_L,), jnp.int32)

    regs = _ladder_pass(xrow, _D)
    t_est = _select_kth(regs, iota)

    def wmain(j, accs):
        a_gt, a_ge = accs
        v = xrow[pl.ds(j * _L, _L)]
        gt = v > t_est
        keep = v >= t_est
        orow[pl.ds(j * _L, _L)] = jnp.where(keep, v, jnp.float32(0.0))
        mrow[pl.ds(j * _L, _L)] = jnp.where(keep, jnp.float32(1.0), jnp.float32(0.0))
        a_gt = a_gt + jnp.where(gt, jnp.int32(1), jnp.int32(0))
        a_ge = a_ge + jnp.where(keep, jnp.int32(1), jnp.int32(0))
        return (a_gt, a_ge)

    a_gt, a_ge = lax.fori_loop(0, _CH, wmain, (zero_i, zero_i), unroll=8)
    c_gt = _butterfly(a_gt, iota, jnp.add)
    c_ge = _butterfly(a_ge, iota, jnp.add)

    def _rewrite(t, m_total):
        def wfix(j, used):
            v = xrow[pl.ds(j * _L, _L)]
            gt = v > t
            eq = v == t
            c = jnp.where(eq, jnp.int32(1), jnp.int32(0))
            pre = c
            for st in (1, 2, 4, 8):
                shifted = jnp.take(pre, jnp.maximum(iota - st, 0))
                pre = pre + jnp.where(iota >= st, shifted, jnp.int32(0))
            keep = jnp.logical_or(gt, jnp.logical_and(eq, (used + pre) <= m_total))
            orow[pl.ds(j * _L, _L)] = jnp.where(keep, v, jnp.float32(0.0))
            mrow[pl.ds(j * _L, _L)] = jnp.where(keep, jnp.float32(1.0), jnp.float32(0.0))
            return used + _butterfly(c, iota, jnp.add)

        lax.fori_loop(0, _CH, wfix, zero_i)

    ok = c_gt[0] <= jnp.int32(_K - 1)
    nfb = jnp.where(ok, jnp.int32(0), jnp.int32(1))

    def fb(_, carry):
        regs64 = _ladder_pass(xrow, _K)
        tf = _select_kth(regs64, iota)
        fb_gt, _unused = _count_pass(xrow, tf, iota)
        _rewrite(tf, jnp.int32(_K) - fb_gt)
        return carry

    lax.fori_loop(0, nfb, fb, jnp.int32(0))

    tie = jnp.logical_and(ok, c_ge[0] > jnp.int32(_K))
    nt = jnp.where(tie, jnp.int32(1), jnp.int32(0))

    def tiefix(_, carry):
        _rewrite(t_est, jnp.int32(_K) - c_gt)
        return carry

    lax.fori_loop(0, nt, tiefix, jnp.int32(0))


def _sc_body(x_hbm, out_hbm, mask_hbm, xrow, orow, mrow):
    c = lax.axis_index("c")
    s = lax.axis_index("s")
    wid = s * _NC + c
    base = wid * _RPW

    def row_step(i, carry):
        r = base + i
        pltpu.sync_copy(x_hbm.at[r], xrow)
        _process_row(xrow, orow, mrow)
        pltpu.sync_copy(orow, out_hbm.at[r])
        pltpu.sync_copy(mrow, mask_hbm.at[r])
        return carry

    lax.fori_loop(0, _RPW, row_step, jnp.int32(0))


def kernel(x):
    mesh = plsc.VectorSubcoreMesh(core_axis_name="c", subcore_axis_name="s")
    f = pl.kernel(
        _sc_body,
        out_type=[
            jax.ShapeDtypeStruct((_B, _N), jnp.float32),
            jax.ShapeDtypeStruct((_B, _N), jnp.float32),
        ],
        mesh=mesh,
        scratch_types=[
            pltpu.VMEM((_N,), jnp.float32),
            pltpu.VMEM((_N,), jnp.float32),
            pltpu.VMEM((_N,), jnp.float32),
        ],
    )
    out, mask = f(x)
    return (out, mask)
